# Initial kernel scaffold; baseline (speedup 1.0000x reference)
import jax
import jax.numpy as jnp
from jax.experimental import pallas as pl


def kernel(q, k, v, s, M_0):
    raise NotImplementedError("write your pallas kernel here")


# speedup vs baseline: 133.2474x; 133.2474x over previous
import functools

import jax
import jax.numpy as jnp
from jax.experimental import pallas as pl
from jax.experimental.pallas import tpu as pltpu


def _la_chunk_kernel(s_ref, q_ref, k_ref, v_ref, m0_ref, o_ref, state_ref,
                     *, chunk, num_seg):
    i = pl.program_id(0)
    c0 = i * chunk

    @pl.when(i == 0)
    def _init():
        state_ref[...] = jnp.zeros_like(state_ref)

    q = q_ref[...]
    k = k_ref[...]
    v = v_ref[...]

    t_col = c0 + jax.lax.broadcasted_iota(jnp.int32, (chunk, 1), 0)
    seg = jnp.full((chunk, 1), -1, dtype=jnp.int32)
    for p in range(num_seg + 1):
        seg = seg + (t_col >= s_ref[p]).astype(jnp.int32)
    valid = (seg >= 0) & (seg < num_seg)
    seg_c = jnp.clip(seg, 0, num_seg - 1)
    start = jnp.zeros((chunk, 1), jnp.int32)
    for p in range(num_seg):
        start = jnp.where(seg_c == p, s_ref[p], start)

    validf = valid.astype(jnp.float32)
    carryf = (valid & (start < c0)).astype(jnp.float32)

    a = jax.lax.dot_general(q, k, (((1,), (1,)), ((), ())),
                            preferred_element_type=jnp.float32)
    causal = t_col >= jax.lax.broadcasted_iota(jnp.int32, (1, chunk), 1) + c0
    same_seg = seg_c == seg_c.reshape(1, chunk)
    mask = (causal & same_seg & valid & valid.reshape(1, chunk))
    a = a * mask.astype(jnp.float32)
    o_intra = jax.lax.dot_general(a, v, (((1,), (0,)), ((), ())),
                                  preferred_element_type=jnp.float32)

    q_m0 = jax.lax.dot_general(q, m0_ref[...], (((1,), (0,)), ((), ())),
                               preferred_element_type=jnp.float32)
    q_s = jax.lax.dot_general(q, state_ref[...], (((1,), (0,)), ((), ())),
                              preferred_element_type=jnp.float32)
    o_ref[...] = validf * q_m0 + carryf * q_s + o_intra

    t_end = c0 + chunk - 1
    seg_end = jnp.int32(-1)
    for p in range(num_seg + 1):
        seg_end = seg_end + (t_end >= s_ref[p]).astype(jnp.int32)
    seg_end_c = jnp.clip(seg_end, 0, num_seg - 1)
    start_end = jnp.int32(0)
    for p in range(num_seg):
        start_end = jnp.where(seg_end_c == p, s_ref[p], start_end)
    keep = (start_end < c0).astype(jnp.float32)

    suffix = (valid & (seg_c == seg_end_c)).astype(jnp.float32)
    k_m = k * suffix
    s_new = jax.lax.dot_general(k_m, v, (((0,), (0,)), ((), ())),
                                preferred_element_type=jnp.float32)
    state_ref[...] = keep * state_ref[...] + s_new


def kernel(q, k, v, s, M_0):
    T, d = q.shape
    num_seg = s.shape[0] - 1
    chunk = 256
    grid = T // chunk

    fn = functools.partial(_la_chunk_kernel, chunk=chunk, num_seg=num_seg)
    return pl.pallas_call(
        fn,
        grid_spec=pltpu.PrefetchScalarGridSpec(
            num_scalar_prefetch=1,
            grid=(grid,),
            in_specs=[
                pl.BlockSpec((chunk, d), lambda i, s_ref: (i, 0)),
                pl.BlockSpec((chunk, d), lambda i, s_ref: (i, 0)),
                pl.BlockSpec((chunk, d), lambda i, s_ref: (i, 0)),
                pl.BlockSpec((d, d), lambda i, s_ref: (0, 0)),
            ],
            out_specs=pl.BlockSpec((chunk, d), lambda i, s_ref: (i, 0)),
            scratch_shapes=[pltpu.VMEM((d, d), jnp.float32)],
        ),
        out_shape=jax.ShapeDtypeStruct((T, d), jnp.float32),
        compiler_params=pltpu.CompilerParams(
            dimension_semantics=("arbitrary",),
        ),
    )(s, q, k, v, M_0)


# speedup vs baseline: 148.5015x; 1.1145x over previous
import functools

import jax
import jax.numpy as jnp
from jax.experimental import pallas as pl
from jax.experimental.pallas import tpu as pltpu


def _la_chunk_kernel(s_ref, q_ref, k_ref, v_ref, m0_ref, o_ref, state_ref,
                     *, chunk, num_seg):
    i = pl.program_id(0)
    c0 = i * chunk

    @pl.when(i == 0)
    def _init():
        state_ref[...] = jnp.zeros_like(state_ref)

    q = q_ref[...]
    k = k_ref[...]
    v = v_ref[...]

    t_col = c0 + jax.lax.broadcasted_iota(jnp.int32, (chunk, 1), 0)
    seg = jnp.full((chunk, 1), -1, dtype=jnp.int32)
    for p in range(num_seg + 1):
        seg = seg + (t_col >= s_ref[p]).astype(jnp.int32)
    valid = (seg >= 0) & (seg < num_seg)
    seg_c = jnp.clip(seg, 0, num_seg - 1)
    start = jnp.zeros((chunk, 1), jnp.int32)
    for p in range(num_seg):
        start = jnp.where(seg_c == p, s_ref[p], start)

    validf = valid.astype(jnp.float32)
    carryf = (valid & (start < c0)).astype(jnp.float32)

    a = jax.lax.dot_general(q, k, (((1,), (1,)), ((), ())),
                            preferred_element_type=jnp.float32)
    causal = t_col >= jax.lax.broadcasted_iota(jnp.int32, (1, chunk), 1) + c0
    same_seg = seg_c == seg_c.reshape(1, chunk)
    mask = (causal & same_seg & valid & valid.reshape(1, chunk))
    a = a * mask.astype(jnp.float32)
    o_intra = jax.lax.dot_general(a, v, (((1,), (0,)), ((), ())),
                                  preferred_element_type=jnp.float32)

    q_m0 = jax.lax.dot_general(q, m0_ref[...], (((1,), (0,)), ((), ())),
                               preferred_element_type=jnp.float32)
    q_s = jax.lax.dot_general(q, state_ref[...], (((1,), (0,)), ((), ())),
                              preferred_element_type=jnp.float32)
    o_ref[...] = validf * q_m0 + carryf * q_s + o_intra

    t_end = c0 + chunk - 1
    seg_end = jnp.int32(-1)
    for p in range(num_seg + 1):
        seg_end = seg_end + (t_end >= s_ref[p]).astype(jnp.int32)
    seg_end_c = jnp.clip(seg_end, 0, num_seg - 1)
    start_end = jnp.int32(0)
    for p in range(num_seg):
        start_end = jnp.where(seg_end_c == p, s_ref[p], start_end)
    keep = (start_end < c0).astype(jnp.float32)

    suffix = (valid & (seg_c == seg_end_c)).astype(jnp.float32)
    k_m = k * suffix
    s_new = jax.lax.dot_general(k_m, v, (((0,), (0,)), ((), ())),
                                preferred_element_type=jnp.float32)
    state_ref[...] = keep * state_ref[...] + s_new


def kernel(q, k, v, s, M_0):
    T, d = q.shape
    num_seg = s.shape[0] - 1
    chunk = 512
    grid = T // chunk

    fn = functools.partial(_la_chunk_kernel, chunk=chunk, num_seg=num_seg)
    return pl.pallas_call(
        fn,
        grid_spec=pltpu.PrefetchScalarGridSpec(
            num_scalar_prefetch=1,
            grid=(grid,),
            in_specs=[
                pl.BlockSpec((chunk, d), lambda i, s_ref: (i, 0)),
                pl.BlockSpec((chunk, d), lambda i, s_ref: (i, 0)),
                pl.BlockSpec((chunk, d), lambda i, s_ref: (i, 0)),
                pl.BlockSpec((d, d), lambda i, s_ref: (0, 0)),
            ],
            out_specs=pl.BlockSpec((chunk, d), lambda i, s_ref: (i, 0)),
            scratch_shapes=[pltpu.VMEM((d, d), jnp.float32)],
        ),
        out_shape=jax.ShapeDtypeStruct((T, d), jnp.float32),
        compiler_params=pltpu.CompilerParams(
            dimension_semantics=("arbitrary",),
        ),
    )(s, q, k, v, M_0)
